# Initial kernel scaffold; baseline (speedup 1.0000x reference)
import jax
import jax.numpy as jnp
from jax.experimental import pallas as pl


def kernel(patches, pos_table):
    raise NotImplementedError("write your pallas kernel here")


# speedup vs baseline: 1.0459x; 1.0459x over previous
import jax
import jax.numpy as jnp
from jax.experimental import pallas as pl


def _add_kernel(p_ref, t_ref, o_ref):
    o_ref[...] = p_ref[...] + t_ref[...]


def kernel(patches, pos_table):
    B, N, D = patches.shape
    BB = 2
    return pl.pallas_call(
        _add_kernel,
        grid=(B // BB,),
        in_specs=[
            pl.BlockSpec((BB, N, D), lambda b: (b, 0, 0)),
            pl.BlockSpec((N, D), lambda b: (0, 0)),
        ],
        out_specs=pl.BlockSpec((BB, N, D), lambda b: (b, 0, 0)),
        out_shape=jax.ShapeDtypeStruct((B, N, D), patches.dtype),
    )(patches, pos_table)


# speedup vs baseline: 1.0588x; 1.0123x over previous
import jax
import jax.numpy as jnp
from jax.experimental import pallas as pl


def _add_kernel(p_ref, t_ref, o_ref):
    o_ref[...] = p_ref[...] + t_ref[...]


def kernel(patches, pos_table):
    B, N, D = patches.shape
    BB = 4
    return pl.pallas_call(
        _add_kernel,
        grid=(B // BB,),
        in_specs=[
            pl.BlockSpec((BB, N, D), lambda b: (b, 0, 0)),
            pl.BlockSpec((N, D), lambda b: (0, 0)),
        ],
        out_specs=pl.BlockSpec((BB, N, D), lambda b: (b, 0, 0)),
        out_shape=jax.ShapeDtypeStruct((B, N, D), patches.dtype),
    )(patches, pos_table)
